# Initial kernel scaffold; baseline (speedup 1.0000x reference)
import jax
import jax.numpy as jnp
from jax import lax
from jax.experimental import pallas as pl
from jax.experimental.pallas import tpu as pltpu
from jax.experimental.pallas import tpu_sc as plsc

N_NODES = 10000
E_EDGES = 320000
D = 128
D_EDGE = 16

NC = 2
NS = 16
NW = NC * NS

LANES = 16
ROW = 128
ROWS_PAD = 2560
RPT = ROWS_PAD // NW
SINK = N_NODES
N_ACC = 10016
NPT_ACC = N_ACC // NS
NPT_OUT = N_NODES // NS


def _sc_body(x_a_h, x_b_h,
             src_ab_h, dst_ab_h, attr_ab_h,
             src_ba_h, dst_ba_h, attr_ba_h,
             s_ab_o, t_ab_o, c_ab_o, s_ba_o, t_ba_o, c_ba_o,
             S_sh, T_sh, src_v, dst_v, rows_v, attr_v, cnt_v, zb_t, sem):
  c = lax.axis_index("c")
  s = lax.axis_index("s")
  wid = c * NS + s

  zf = jnp.zeros((LANES,), jnp.float32)
  zi = jnp.zeros((LANES,), jnp.int32)
  ones_i = jnp.full((LANES,), 1, jnp.int32)

  def _zb(i, _):
    zb_t[i, :] = zf
    return 0
  lax.fori_loop(0, NPT_ACC, _zb, 0)

  def _run_relation(x_h, src_h, dst_h, attr_h, s_o, t_o, c_o):
    def _zr(k, _):
      rows_v[k // 8, pl.ds((k % 8) * LANES, LANES)] = zf
      return 0
    lax.fori_loop(0, 128 * 8, _zr, 0)

    def _zc(k, _):
      cnt_v[pl.ds(k * LANES, LANES)] = zi
      return 0
    lax.fori_loop(0, N_ACC // LANES, _zc, 0)

    base = s * NPT_ACC
    for off in (0, 128, 256, 384):
      pltpu.sync_copy(rows_v, S_sh.at[pl.ds(base + off, 128)])
    pltpu.sync_copy(rows_v.at[pl.ds(0, NPT_ACC - 512)],
                    S_sh.at[pl.ds(base + 512, NPT_ACC - 512)])
    pltpu.sync_copy(zb_t, T_sh.at[pl.ds(base, NPT_ACC)])
    plsc.subcore_barrier()

    r0 = wid * RPT
    pltpu.sync_copy(src_h.at[pl.ds(r0, RPT)], src_v)
    pltpu.sync_copy(dst_h.at[pl.ds(r0, RPT)], dst_v)

    def _cnt(k, _):
      idx = dst_v[k // 8, pl.ds((k % 8) * LANES, LANES)]
      plsc.addupdate_scatter(cnt_v, [idx], ones_i)
      return 0
    lax.fori_loop(0, RPT * 8, _cnt, 0)

    def _gs(k, _):
      pltpu.async_copy(x_h.at[src_v.at[k]], rows_v, sem).wait()
      pltpu.sync_copy(rows_v, S_sh.at[dst_v.at[k]], add=True)
      return 0
    lax.fori_loop(0, RPT, _gs, 0)

    def _att(cix, _):
      e0 = (r0 + cix * 8) * ROW
      pltpu.sync_copy(attr_h.at[pl.ds(e0, 8 * ROW)], attr_v)
      for j in range(8):
        pltpu.sync_copy(attr_v.at[pl.ds(j * ROW, ROW)],
                        T_sh.at[dst_v.at[cix * 8 + j]], add=True)
      return 0
    lax.fori_loop(0, RPT // 8, _att, 0)

    plsc.subcore_barrier()

    o0 = s * NPT_OUT
    pltpu.sync_copy(S_sh.at[pl.ds(o0, NPT_OUT)], s_o.at[c, pl.ds(o0, NPT_OUT)])
    pltpu.sync_copy(T_sh.at[pl.ds(o0, NPT_OUT)], t_o.at[c, pl.ds(o0, NPT_OUT)])
    pltpu.sync_copy(cnt_v.at[pl.ds(0, N_NODES)], c_o.at[wid])
    plsc.subcore_barrier()

  _run_relation(x_a_h, src_ab_h, dst_ab_h, attr_ab_h, s_ab_o, t_ab_o, c_ab_o)
  _run_relation(x_b_h, src_ba_h, dst_ba_h, attr_ba_h, s_ba_o, t_ba_o, c_ba_o)


def _sc_segsums(x_a, x_b, src_ab, dst_ab, attr_ab, src_ba, dst_ba, attr_ba):
  mesh = plsc.VectorSubcoreMesh(core_axis_name="c", subcore_axis_name="s")
  f32 = jnp.float32
  out_type = (
      jax.ShapeDtypeStruct((NC, N_NODES, D), f32),
      jax.ShapeDtypeStruct((NC, N_NODES, D_EDGE), f32),
      jax.ShapeDtypeStruct((NW, N_NODES), jnp.int32),
      jax.ShapeDtypeStruct((NC, N_NODES, D), f32),
      jax.ShapeDtypeStruct((NC, N_NODES, D_EDGE), f32),
      jax.ShapeDtypeStruct((NW, N_NODES), jnp.int32),
  )
  scratch = [
      pltpu.VMEM_SHARED((N_ACC, D), f32),
      pltpu.VMEM_SHARED((N_ACC, D_EDGE), f32),
      pltpu.VMEM((RPT, ROW), jnp.int32),
      pltpu.VMEM((RPT, ROW), jnp.int32),
      pltpu.VMEM((128, D), f32),
      pltpu.VMEM((8 * ROW, D_EDGE), f32),
      pltpu.VMEM((N_ACC,), jnp.int32),
      pltpu.VMEM((NPT_ACC, D_EDGE), f32),
      pltpu.SemaphoreType.DMA,
  ]
  return pl.kernel(
      _sc_body, out_type=out_type, mesh=mesh, scratch_types=scratch,
  )(x_a, x_b, src_ab, dst_ab, attr_ab, src_ba, dst_ba, attr_ba)


BN = 2000


def _dense_body(x_ref, s0_ref, s1_ref, t0_ref, t1_ref, c_ref,
                wn_top, wn_bot, bn_r, ws_r, bs_r, wu_top, wu_bot, bu_r,
                wsf_r, bsf_r, out_ref):
  hi = jax.lax.Precision.HIGHEST
  x = x_ref[...]
  S = s0_ref[...] + s1_ref[...]
  T = t0_ref[...] + t1_ref[...]
  cnt = jnp.sum(c_ref[...].astype(jnp.float32), axis=0)
  summed = (jnp.dot(S, wn_top[...], precision=hi)
            + jnp.dot(T, wn_bot[...], precision=hi)
            + cnt[:, None] * bn_r[...])
  agg = summed / jnp.maximum(cnt, 1.0)[:, None]
  self_t = jnp.dot(x, ws_r[...], precision=hi) + bs_r[...]
  m = (jnp.dot(self_t, wu_top[...], precision=hi)
       + jnp.dot(agg, wu_bot[...], precision=hi) + bu_r[...])
  out_ref[...] = m + jnp.dot(x, wsf_r[...], precision=hi) + bsf_r[...]


def _dense(x, s_part, t_part, c_part, wn_top, wn_bot, bn, ws, bs,
           wu_top, wu_bot, bu, wsf, bsf):
  n = x.shape[0]
  grid = (n // BN,)
  row_spec = lambda width: pl.BlockSpec((BN, width), lambda i: (i, 0))
  full = lambda a: pl.BlockSpec(a.shape, lambda i: (0,) * a.ndim)
  return pl.pallas_call(
      _dense_body,
      grid=grid,
      in_specs=[
          row_spec(D), row_spec(D), row_spec(D),
          row_spec(D_EDGE), row_spec(D_EDGE),
          pl.BlockSpec((NW, BN), lambda i: (0, i)),
          full(wn_top), full(wn_bot), full(bn), full(ws), full(bs),
          full(wu_top), full(wu_bot), full(bu), full(wsf), full(bsf),
      ],
      out_specs=row_spec(D),
      out_shape=jax.ShapeDtypeStruct((n, D), jnp.float32),
  )(x, s_part[0], s_part[1], t_part[0], t_part[1], c_part,
    wn_top, wn_bot, bn, ws, bs, wu_top, wu_bot, bu, wsf, bsf)


def _pad_edges(edge_index, edge_attr):
  src = edge_index[0]
  dst = edge_index[1]
  pad = ROWS_PAD * ROW - E_EDGES
  src = jnp.concatenate([src, jnp.zeros((pad,), jnp.int32)]).reshape(ROWS_PAD, ROW)
  dst = jnp.concatenate([dst, jnp.full((pad,), SINK, jnp.int32)]).reshape(ROWS_PAD, ROW)
  attr = jnp.concatenate([edge_attr, jnp.zeros((pad, D_EDGE), jnp.float32)])
  return src, dst, attr


@jax.jit
def kernel(x_a, x_b, edge_index_ab, edge_index_ba, edge_attr_ab, edge_attr_ba,
           W_neigh_ab, b_neigh_ab, W_self_ab, b_self_ab, W_update_ab, b_update_ab,
           W_neigh_ba, b_neigh_ba, W_self_ba, b_self_ba, W_update_ba, b_update_ba,
           W_sf_a, b_sf_a, W_sf_b, b_sf_b):
  src_ab, dst_ab, attr_ab = _pad_edges(edge_index_ab, edge_attr_ab)
  src_ba, dst_ba, attr_ba = _pad_edges(edge_index_ba, edge_attr_ba)

  s_ab, t_ab, c_ab, s_ba, t_ba, c_ba = _sc_segsums(
      x_a, x_b, src_ab, dst_ab, attr_ab, src_ba, dst_ba, attr_ba)

  def two_d(b):
    return b.reshape(1, D)

  out_a = _dense(x_a, s_ba, t_ba, c_ba,
                 W_neigh_ba[:D], W_neigh_ba[D:], two_d(b_neigh_ba),
                 W_self_ba, two_d(b_self_ba),
                 W_update_ba[:D], W_update_ba[D:], two_d(b_update_ba),
                 W_sf_a, two_d(b_sf_a))
  out_b = _dense(x_b, s_ab, t_ab, c_ab,
                 W_neigh_ab[:D], W_neigh_ab[D:], two_d(b_neigh_ab),
                 W_self_ab, two_d(b_self_ab),
                 W_update_ab[:D], W_update_ab[D:], two_d(b_update_ab),
                 W_sf_b, two_d(b_sf_b))
  return (out_a, out_b)


# speedup vs baseline: 2.2517x; 2.2517x over previous
import jax
import jax.numpy as jnp
from jax import lax
from jax.experimental import pallas as pl
from jax.experimental.pallas import tpu as pltpu
from jax.experimental.pallas import tpu_sc as plsc

N_NODES = 10000
E_EDGES = 320000
D = 128
D_EDGE = 16

NC = 2
NS = 16
NW = NC * NS

LANES = 16
ROW = 128
ROWS_PAD = 2560
RPT = ROWS_PAD // NW
CHUNK_ROWS = 16
SINK = N_NODES
N_ACC = 10240
NPT_ACC = N_ACC // NS


def _sc_body(x_a_h, x_b_h,
             src_ab_h, dst_ab_h, attr_ab_h,
             src_ba_h, dst_ba_h, attr_ba_h,
             s_ab_o, t_ab_o, c_ab_o, s_ba_o, t_ba_o, c_ba_o,
             S_sh, T_sh, src_v, dst_v, rows_v, attr_v, cnt_v, zb_t, sem):
  c = lax.axis_index("c")
  s = lax.axis_index("s")
  wid = c * NS + s

  zf = jnp.zeros((LANES,), jnp.float32)
  zi = jnp.zeros((LANES,), jnp.int32)
  ones_i = jnp.full((LANES,), 1, jnp.int32)

  def _zb(i, _):
    zb_t[i, :] = zf
    return 0
  lax.fori_loop(0, 128, _zb, 0)

  def _run_relation(x_h, src_h, dst_h, attr_h, s_o, t_o, c_o):
    def _zr(k, _):
      rows_v[k // 8, pl.ds((k % 8) * LANES, LANES)] = zf
      return 0
    lax.fori_loop(0, 128 * 8, _zr, 0)

    def _zc(k, _):
      cnt_v[pl.ds(k * LANES, LANES)] = zi
      return 0
    lax.fori_loop(0, N_ACC // LANES, _zc, 0)

    base = s * NPT_ACC
    for off in range(0, NPT_ACC, 128):
      pltpu.sync_copy(rows_v, S_sh.at[pl.ds(base + off, 128)])
      pltpu.sync_copy(zb_t, T_sh.at[pl.ds(base + off, 128)])
    plsc.subcore_barrier()

    def _chunk(ci, _):
      r0 = wid * RPT + ci * CHUNK_ROWS
      pltpu.sync_copy(src_h.at[pl.ds(r0, CHUNK_ROWS)], src_v)
      pltpu.sync_copy(dst_h.at[pl.ds(r0, CHUNK_ROWS)], dst_v)

      def _cnt(k, _):
        idx = dst_v[k // 8, pl.ds((k % 8) * LANES, LANES)]
        plsc.addupdate_scatter(cnt_v, [idx], ones_i)
        return 0
      lax.fori_loop(0, CHUNK_ROWS * 8, _cnt, 0)

      def _gs(k, _):
        pltpu.async_copy(x_h.at[src_v.at[k]], rows_v, sem).wait()
        pltpu.sync_copy(rows_v, S_sh.at[dst_v.at[k]], add=True)
        pltpu.sync_copy(attr_h.at[pl.ds((r0 + k) * ROW, ROW)], attr_v)
        pltpu.sync_copy(attr_v, T_sh.at[dst_v.at[k]], add=True)
        return 0
      lax.fori_loop(0, CHUNK_ROWS, _gs, 0)
      return 0
    lax.fori_loop(0, RPT // CHUNK_ROWS, _chunk, 0)

    plsc.subcore_barrier()

    o0 = s * NPT_ACC
    pltpu.sync_copy(S_sh.at[pl.ds(o0, NPT_ACC)], s_o.at[c, pl.ds(o0, NPT_ACC)])
    pltpu.sync_copy(T_sh.at[pl.ds(o0, NPT_ACC)], t_o.at[c, pl.ds(o0, NPT_ACC)])
    pltpu.sync_copy(cnt_v, c_o.at[wid, 0])
    plsc.subcore_barrier()

  _run_relation(x_a_h, src_ab_h, dst_ab_h, attr_ab_h, s_ab_o, t_ab_o, c_ab_o)
  _run_relation(x_b_h, src_ba_h, dst_ba_h, attr_ba_h, s_ba_o, t_ba_o, c_ba_o)


def _sc_segsums(x_a, x_b, src_ab, dst_ab, attr_ab, src_ba, dst_ba, attr_ba):
  mesh = plsc.VectorSubcoreMesh(core_axis_name="c", subcore_axis_name="s")
  f32 = jnp.float32
  out_type = (
      jax.ShapeDtypeStruct((NC, N_ACC, D), f32),
      jax.ShapeDtypeStruct((NC, N_ACC, D_EDGE), f32),
      jax.ShapeDtypeStruct((NW, 1, N_ACC), jnp.int32),
      jax.ShapeDtypeStruct((NC, N_ACC, D), f32),
      jax.ShapeDtypeStruct((NC, N_ACC, D_EDGE), f32),
      jax.ShapeDtypeStruct((NW, 1, N_ACC), jnp.int32),
  )
  scratch = [
      pltpu.VMEM_SHARED((N_ACC, D), f32),
      pltpu.VMEM_SHARED((N_ACC, D_EDGE), f32),
      pltpu.VMEM((CHUNK_ROWS, ROW), jnp.int32),
      pltpu.VMEM((CHUNK_ROWS, ROW), jnp.int32),
      pltpu.VMEM((ROW, D), f32),
      pltpu.VMEM((ROW, D_EDGE), f32),
      pltpu.VMEM((N_ACC,), jnp.int32),
      pltpu.VMEM((128, D_EDGE), f32),
      pltpu.SemaphoreType.DMA,
  ]
  return pl.kernel(
      _sc_body, out_type=out_type, mesh=mesh, scratch_types=scratch,
      compiler_params=pltpu.CompilerParams(
          needs_layout_passes=False, use_tc_tiling_on_sc=False),
  )(x_a, x_b, src_ab, dst_ab, attr_ab, src_ba, dst_ba, attr_ba)


BN = 2000


def _dense_body(x_ref, s0_ref, s1_ref, t0_ref, t1_ref, c_ref,
                wn_top, wn_bot, bn_r, ws_r, bs_r, wu_top, wu_bot, bu_r,
                wsf_r, bsf_r, out_ref):
  hi = jax.lax.Precision.HIGHEST
  x = x_ref[...]
  S = s0_ref[...] + s1_ref[...]
  T = t0_ref[...] + t1_ref[...]
  cnt = jnp.sum(c_ref[...].astype(jnp.float32), axis=1)
  summed = (jnp.dot(S, wn_top[...], precision=hi)
            + jnp.dot(T, wn_bot[...], precision=hi)
            + cnt[:, None] * bn_r[...])
  agg = summed / jnp.maximum(cnt, 1.0)[:, None]
  self_t = jnp.dot(x, ws_r[...], precision=hi) + bs_r[...]
  m = (jnp.dot(self_t, wu_top[...], precision=hi)
       + jnp.dot(agg, wu_bot[...], precision=hi) + bu_r[...])
  out_ref[...] = m + jnp.dot(x, wsf_r[...], precision=hi) + bsf_r[...]


def _dense(x, s_part, t_part, c_part, wn_top, wn_bot, bn, ws, bs,
           wu_top, wu_bot, bu, wsf, bsf):
  n = x.shape[0]
  grid = (n // BN,)
  row_spec = lambda width: pl.BlockSpec((BN, width), lambda i: (i, 0))
  full = lambda a: pl.BlockSpec(a.shape, lambda i: (0,) * a.ndim)
  return pl.pallas_call(
      _dense_body,
      grid=grid,
      in_specs=[
          row_spec(D), row_spec(D), row_spec(D),
          row_spec(D_EDGE), row_spec(D_EDGE),
          pl.BlockSpec((BN, NW), lambda i: (i, 0)),
          full(wn_top), full(wn_bot), full(bn), full(ws), full(bs),
          full(wu_top), full(wu_bot), full(bu), full(wsf), full(bsf),
      ],
      out_specs=row_spec(D),
      out_shape=jax.ShapeDtypeStruct((n, D), jnp.float32),
  )(x, s_part[0], s_part[1], t_part[0], t_part[1],
    c_part.reshape(NW, N_ACC).T,
    wn_top, wn_bot, bn, ws, bs, wu_top, wu_bot, bu, wsf, bsf)


def _pad_edges(edge_index, edge_attr):
  src = edge_index[0]
  dst = edge_index[1]
  pad = ROWS_PAD * ROW - E_EDGES
  src = jnp.concatenate([src, jnp.zeros((pad,), jnp.int32)]).reshape(ROWS_PAD, ROW)
  dst = jnp.concatenate([dst, jnp.full((pad,), SINK, jnp.int32)]).reshape(ROWS_PAD, ROW)
  attr = jnp.concatenate([edge_attr, jnp.zeros((pad, D_EDGE), jnp.float32)])
  return src, dst, attr


@jax.jit
def kernel(x_a, x_b, edge_index_ab, edge_index_ba, edge_attr_ab, edge_attr_ba,
           W_neigh_ab, b_neigh_ab, W_self_ab, b_self_ab, W_update_ab, b_update_ab,
           W_neigh_ba, b_neigh_ba, W_self_ba, b_self_ba, W_update_ba, b_update_ba,
           W_sf_a, b_sf_a, W_sf_b, b_sf_b):
  src_ab, dst_ab, attr_ab = _pad_edges(edge_index_ab, edge_attr_ab)
  src_ba, dst_ba, attr_ba = _pad_edges(edge_index_ba, edge_attr_ba)

  s_ab, t_ab, c_ab, s_ba, t_ba, c_ba = _sc_segsums(
      x_a, x_b, src_ab, dst_ab, attr_ab, src_ba, dst_ba, attr_ba)

  def two_d(b):
    return b.reshape(1, D)

  out_a = _dense(x_a, s_ba, t_ba, c_ba,
                 W_neigh_ba[:D], W_neigh_ba[D:], two_d(b_neigh_ba),
                 W_self_ba, two_d(b_self_ba),
                 W_update_ba[:D], W_update_ba[D:], two_d(b_update_ba),
                 W_sf_a, two_d(b_sf_a))
  out_b = _dense(x_b, s_ab, t_ab, c_ab,
                 W_neigh_ab[:D], W_neigh_ab[D:], two_d(b_neigh_ab),
                 W_self_ab, two_d(b_self_ab),
                 W_update_ab[:D], W_update_ab[D:], two_d(b_update_ab),
                 W_sf_b, two_d(b_sf_b))
  return (out_a, out_b)


# speedup vs baseline: 2.4958x; 1.1084x over previous
import jax
import jax.numpy as jnp
from jax import lax
from jax.experimental import pallas as pl
from jax.experimental.pallas import tpu as pltpu
from jax.experimental.pallas import tpu_sc as plsc

N_NODES = 10000
E_EDGES = 320000
D = 128
D_EDGE = 16

NC = 2
NS = 16
NW = NC * NS

LANES = 16
G = 64
NB = 10240 // G
NBC = 16
NCH = NB // NBC
ROWS_PAD = 2560
EPT = 10240
SINK = N_NODES
N_ACC = 10240
NPT_ACC = N_ACC // NS


def _sc_body(x_a_h, x_b_h,
             src_ab_h, dst_ab_h, attr_ab_h,
             src_ba_h, dst_ba_h, attr_ba_h,
             s_ab_o, t_ab_o, c_ab_o, s_ba_o, t_ba_o, c_ba_o,
             S_sh, T_sh, src_v, dst_v, rows0, rows1, attr0, attr1, cnt_v,
             zb_t, gsem0, gsem1, ssem0, ssem1, asem0, asem1, tsem0, tsem1):
  c = lax.axis_index("c")
  s = lax.axis_index("s")
  wid = c * NS + s

  zf = jnp.zeros((LANES,), jnp.float32)
  zi = jnp.zeros((LANES,), jnp.int32)
  ones_i = jnp.full((LANES,), 1, jnp.int32)

  def _zb(i, _):
    zb_t[i, :] = zf
    return 0
  lax.fori_loop(0, 128, _zb, 0)

  def _run_relation(x_h, src_h, dst_h, attr_h, s_o, t_o, c_o):
    def _zr(k, _):
      rows0[k // 8, pl.ds((k % 8) * LANES, LANES)] = zf
      return 0
    lax.fori_loop(0, G * 8, _zr, 0)

    def _zc(k, _):
      cnt_v[pl.ds(k * LANES, LANES)] = zi
      return 0
    lax.fori_loop(0, N_ACC // LANES, _zc, 0)

    base = s * NPT_ACC
    for off in range(0, NPT_ACC, G):
      pltpu.sync_copy(rows0, S_sh.at[pl.ds(base + off, G)])
    for off in range(0, NPT_ACC, 128):
      pltpu.sync_copy(zb_t, T_sh.at[pl.ds(base + off, 128)])
    plsc.subcore_barrier()

    e_tile = wid * EPT

    def _chunk(ci, _):
      b_chunk = wid * NB + ci * NBC
      e_chunk = b_chunk * G
      pltpu.sync_copy(src_h.at[pl.ds(b_chunk, NBC)], src_v)
      pltpu.sync_copy(dst_h.at[pl.ds(b_chunk, NBC)], dst_v)

      pltpu.async_copy(x_h.at[src_v.at[0]], rows0, gsem0)
      pltpu.async_copy(x_h.at[src_v.at[1]], rows1, gsem1)
      pltpu.async_copy(attr_h.at[pl.ds(e_chunk, G)], attr0, asem0)
      pltpu.async_copy(attr_h.at[pl.ds(e_chunk + G, G)], attr1, asem1)

      def _pair(p, _):
        b0 = 2 * p
        b1 = b0 + 1
        pltpu.make_async_copy(x_h.at[src_v.at[b0]], rows0, gsem0).wait()
        s0 = pltpu.async_copy(rows0, S_sh.at[dst_v.at[b0]], ssem0, add=True)
        pltpu.make_async_copy(
            attr_h.at[pl.ds(e_chunk + b0 * G, G)], attr0, asem0).wait()
        t0 = pltpu.async_copy(attr0, T_sh.at[dst_v.at[b0]], tsem0, add=True)
        pltpu.make_async_copy(x_h.at[src_v.at[b1]], rows1, gsem1).wait()
        s1 = pltpu.async_copy(rows1, S_sh.at[dst_v.at[b1]], ssem1, add=True)
        pltpu.make_async_copy(
            attr_h.at[pl.ds(e_chunk + b1 * G, G)], attr1, asem1).wait()
        t1 = pltpu.async_copy(attr1, T_sh.at[dst_v.at[b1]], tsem1, add=True)

        for bb, row in ((b0, 0), (b1, 1)):
          def _cnt(j, _, bb=bb):
            idx = dst_v[bb, pl.ds(j * LANES, LANES)]
            plsc.addupdate_scatter(cnt_v, [idx], ones_i)
            return 0
          lax.fori_loop(0, G // LANES, _cnt, 0)

        s0.wait()
        t0.wait()

        @pl.when(p < NBC // 2 - 1)
        def _refill0():
          pltpu.async_copy(x_h.at[src_v.at[b0 + 2]], rows0, gsem0)
          pltpu.async_copy(
              attr_h.at[pl.ds(e_chunk + (b0 + 2) * G, G)], attr0, asem0)

        s1.wait()
        t1.wait()

        @pl.when(p < NBC // 2 - 1)
        def _refill1():
          pltpu.async_copy(x_h.at[src_v.at[b1 + 2]], rows1, gsem1)
          pltpu.async_copy(
              attr_h.at[pl.ds(e_chunk + (b1 + 2) * G, G)], attr1, asem1)

        return 0
      lax.fori_loop(0, NBC // 2, _pair, 0)
      return 0
    lax.fori_loop(0, NCH, _chunk, 0)

    plsc.subcore_barrier()

    o0 = s * NPT_ACC
    pltpu.sync_copy(S_sh.at[pl.ds(o0, NPT_ACC)], s_o.at[c, pl.ds(o0, NPT_ACC)])
    pltpu.sync_copy(T_sh.at[pl.ds(o0, NPT_ACC)], t_o.at[c, pl.ds(o0, NPT_ACC)])
    pltpu.sync_copy(cnt_v, c_o.at[wid, 0])
    plsc.subcore_barrier()

  _run_relation(x_a_h, src_ab_h, dst_ab_h, attr_ab_h, s_ab_o, t_ab_o, c_ab_o)
  _run_relation(x_b_h, src_ba_h, dst_ba_h, attr_ba_h, s_ba_o, t_ba_o, c_ba_o)


def _sc_segsums(x_a, x_b, src_ab, dst_ab, attr_ab, src_ba, dst_ba, attr_ba):
  mesh = plsc.VectorSubcoreMesh(core_axis_name="c", subcore_axis_name="s")
  f32 = jnp.float32
  out_type = (
      jax.ShapeDtypeStruct((NC, N_ACC, D), f32),
      jax.ShapeDtypeStruct((NC, N_ACC, D_EDGE), f32),
      jax.ShapeDtypeStruct((NW, 1, N_ACC), jnp.int32),
      jax.ShapeDtypeStruct((NC, N_ACC, D), f32),
      jax.ShapeDtypeStruct((NC, N_ACC, D_EDGE), f32),
      jax.ShapeDtypeStruct((NW, 1, N_ACC), jnp.int32),
  )
  scratch = [
      pltpu.VMEM_SHARED((N_ACC, D), f32),
      pltpu.VMEM_SHARED((N_ACC, D_EDGE), f32),
      pltpu.VMEM((NBC, G), jnp.int32),
      pltpu.VMEM((NBC, G), jnp.int32),
      pltpu.VMEM((G, D), f32),
      pltpu.VMEM((G, D), f32),
      pltpu.VMEM((G, D_EDGE), f32),
      pltpu.VMEM((G, D_EDGE), f32),
      pltpu.VMEM((N_ACC,), jnp.int32),
      pltpu.VMEM((128, D_EDGE), f32),
  ] + [pltpu.SemaphoreType.DMA] * 8
  return pl.kernel(
      _sc_body, out_type=out_type, mesh=mesh, scratch_types=scratch,
      compiler_params=pltpu.CompilerParams(
          needs_layout_passes=False, use_tc_tiling_on_sc=False),
  )(x_a, x_b, src_ab, dst_ab, attr_ab, src_ba, dst_ba, attr_ba)


BN = 2000


def _dense_body(x_ref, s0_ref, s1_ref, t0_ref, t1_ref, c_ref,
                wn_top, wn_bot, bn_r, ws_r, bs_r, wu_top, wu_bot, bu_r,
                wsf_r, bsf_r, out_ref):
  hi = jax.lax.Precision.HIGHEST
  x = x_ref[...]
  S = s0_ref[...] + s1_ref[...]
  T = t0_ref[...] + t1_ref[...]
  cnt = jnp.sum(c_ref[...].astype(jnp.float32), axis=1)
  summed = (jnp.dot(S, wn_top[...], precision=hi)
            + jnp.dot(T, wn_bot[...], precision=hi)
            + cnt[:, None] * bn_r[...])
  agg = summed / jnp.maximum(cnt, 1.0)[:, None]
  self_t = jnp.dot(x, ws_r[...], precision=hi) + bs_r[...]
  m = (jnp.dot(self_t, wu_top[...], precision=hi)
       + jnp.dot(agg, wu_bot[...], precision=hi) + bu_r[...])
  out_ref[...] = m + jnp.dot(x, wsf_r[...], precision=hi) + bsf_r[...]


def _dense(x, s_part, t_part, c_part, wn_top, wn_bot, bn, ws, bs,
           wu_top, wu_bot, bu, wsf, bsf):
  n = x.shape[0]
  grid = (n // BN,)
  row_spec = lambda width: pl.BlockSpec((BN, width), lambda i: (i, 0))
  full = lambda a: pl.BlockSpec(a.shape, lambda i: (0,) * a.ndim)
  return pl.pallas_call(
      _dense_body,
      grid=grid,
      in_specs=[
          row_spec(D), row_spec(D), row_spec(D),
          row_spec(D_EDGE), row_spec(D_EDGE),
          pl.BlockSpec((BN, NW), lambda i: (i, 0)),
          full(wn_top), full(wn_bot), full(bn), full(ws), full(bs),
          full(wu_top), full(wu_bot), full(bu), full(wsf), full(bsf),
      ],
      out_specs=row_spec(D),
      out_shape=jax.ShapeDtypeStruct((n, D), jnp.float32),
  )(x, s_part[0], s_part[1], t_part[0], t_part[1],
    c_part.reshape(NW, N_ACC).T,
    wn_top, wn_bot, bn, ws, bs, wu_top, wu_bot, bu, wsf, bsf)


def _pad_edges(edge_index, edge_attr):
  src = edge_index[0]
  dst = edge_index[1]
  e_pad = EPT * NW
  pad = e_pad - E_EDGES
  src = jnp.concatenate([src, jnp.zeros((pad,), jnp.int32)]).reshape(e_pad // G, G)
  dst = jnp.concatenate([dst, jnp.full((pad,), SINK, jnp.int32)]).reshape(e_pad // G, G)
  attr = jnp.concatenate([edge_attr, jnp.zeros((pad, D_EDGE), jnp.float32)])
  return src, dst, attr


@jax.jit
def kernel(x_a, x_b, edge_index_ab, edge_index_ba, edge_attr_ab, edge_attr_ba,
           W_neigh_ab, b_neigh_ab, W_self_ab, b_self_ab, W_update_ab, b_update_ab,
           W_neigh_ba, b_neigh_ba, W_self_ba, b_self_ba, W_update_ba, b_update_ba,
           W_sf_a, b_sf_a, W_sf_b, b_sf_b):
  src_ab, dst_ab, attr_ab = _pad_edges(edge_index_ab, edge_attr_ab)
  src_ba, dst_ba, attr_ba = _pad_edges(edge_index_ba, edge_attr_ba)

  s_ab, t_ab, c_ab, s_ba, t_ba, c_ba = _sc_segsums(
      x_a, x_b, src_ab, dst_ab, attr_ab, src_ba, dst_ba, attr_ba)

  def two_d(b):
    return b.reshape(1, D)

  out_a = _dense(x_a, s_ba, t_ba, c_ba,
                 W_neigh_ba[:D], W_neigh_ba[D:], two_d(b_neigh_ba),
                 W_self_ba, two_d(b_self_ba),
                 W_update_ba[:D], W_update_ba[D:], two_d(b_update_ba),
                 W_sf_a, two_d(b_sf_a))
  out_b = _dense(x_b, s_ab, t_ab, c_ab,
                 W_neigh_ab[:D], W_neigh_ab[D:], two_d(b_neigh_ab),
                 W_self_ab, two_d(b_self_ab),
                 W_update_ab[:D], W_update_ab[D:], two_d(b_update_ab),
                 W_sf_b, two_d(b_sf_b))
  return (out_a, out_b)
